# baseline (speedup 1.0000x reference)
import jax
import jax.numpy as jnp
from jax.experimental import pallas as pl
from jax.experimental.pallas import tpu as pltpu

_TILE = 512
_NBUF = 4
_NTOK = 8192
_NBLK = _NTOK // _TILE


def _in_copy(i, x_hbm, xbuf, insem):
    slot = i % _NBUF
    return pltpu.make_async_copy(
        x_hbm.at[pl.ds(i * _TILE, _TILE), :], xbuf.at[slot], insem.at[slot])


def _out_copy(i, obuf, o_hbm, outsem):
    slot = i % _NBUF
    return pltpu.make_async_copy(
        obuf.at[slot], o_hbm.at[pl.ds(i * _TILE, _TILE), :], outsem.at[slot])


def _gating_kernel(x_hbm, w_ref, b_ref, o_hbm, xbuf, obuf, insem, outsem):
    for i in range(_NBUF):
        _in_copy(i, x_hbm, xbuf, insem).start()
    for i in range(_NBLK):
        slot = i % _NBUF
        _in_copy(i, x_hbm, xbuf, insem).wait()
        if i >= _NBUF:
            _out_copy(i - _NBUF, obuf, o_hbm, outsem).wait()
        logits = jax.lax.dot_general(
            xbuf[slot], w_ref[...],
            dimension_numbers=(((1,), (1,)), ((), ())),
            preferred_element_type=jnp.float32,
        )
        logits = logits + b_ref[...]
        m = jnp.max(logits, axis=-1, keepdims=True)
        e = jnp.exp(logits - m)
        s = jnp.sum(e, axis=-1, keepdims=True)
        obuf[slot] = e / s
        _out_copy(i, obuf, o_hbm, outsem).start()
        if i + _NBUF < _NBLK:
            _in_copy(i + _NBUF, x_hbm, xbuf, insem).start()
    for i in range(_NBLK - _NBUF, _NBLK):
        _out_copy(i, obuf, o_hbm, outsem).wait()


def kernel(x, W, b):
    n_tokens, input_dim = x.shape
    num_experts = W.shape[0]
    b2 = b.reshape(1, num_experts)
    return pl.pallas_call(
        _gating_kernel,
        in_specs=[
            pl.BlockSpec(memory_space=pltpu.MemorySpace.HBM),
            pl.BlockSpec(memory_space=pltpu.MemorySpace.VMEM),
            pl.BlockSpec(memory_space=pltpu.MemorySpace.VMEM),
        ],
        out_specs=pl.BlockSpec(memory_space=pltpu.MemorySpace.HBM),
        out_shape=jax.ShapeDtypeStruct((n_tokens, num_experts), jnp.float32),
        scratch_shapes=[
            pltpu.VMEM((_NBUF, _TILE, input_dim), jnp.float32),
            pltpu.VMEM((_NBUF, _TILE, num_experts), jnp.float32),
            pltpu.SemaphoreType.DMA((_NBUF,)),
            pltpu.SemaphoreType.DMA((_NBUF,)),
        ],
    )(x, W, b2)


# speedup vs baseline: 1.0028x; 1.0028x over previous
import jax
import jax.numpy as jnp
from jax.experimental import pallas as pl
from jax.experimental.pallas import tpu as pltpu

_TILE = 1024
_NBUF = 4
_NTOK = 8192
_NBLK = _NTOK // _TILE


def _in_copy(i, x_hbm, xbuf, insem):
    slot = i % _NBUF
    return pltpu.make_async_copy(
        x_hbm.at[pl.ds(i * _TILE, _TILE), :], xbuf.at[slot], insem.at[slot])


def _out_copy(i, obuf, o_hbm, outsem):
    slot = i % _NBUF
    return pltpu.make_async_copy(
        obuf.at[slot], o_hbm.at[pl.ds(i * _TILE, _TILE), :], outsem.at[slot])


def _gating_kernel(x_hbm, w_ref, b_ref, o_hbm, xbuf, obuf, insem, outsem):
    for i in range(_NBUF):
        _in_copy(i, x_hbm, xbuf, insem).start()
    for i in range(_NBLK):
        slot = i % _NBUF
        _in_copy(i, x_hbm, xbuf, insem).wait()
        if i >= _NBUF:
            _out_copy(i - _NBUF, obuf, o_hbm, outsem).wait()
        logits = jax.lax.dot_general(
            xbuf[slot], w_ref[...],
            dimension_numbers=(((1,), (1,)), ((), ())),
            preferred_element_type=jnp.float32,
        )
        logits = logits + b_ref[...]
        m = jnp.max(logits, axis=-1, keepdims=True)
        e = jnp.exp(logits - m)
        s = jnp.sum(e, axis=-1, keepdims=True)
        obuf[slot] = e / s
        _out_copy(i, obuf, o_hbm, outsem).start()
        if i + _NBUF < _NBLK:
            _in_copy(i + _NBUF, x_hbm, xbuf, insem).start()
    for i in range(_NBLK - _NBUF, _NBLK):
        _out_copy(i, obuf, o_hbm, outsem).wait()


def kernel(x, W, b):
    n_tokens, input_dim = x.shape
    num_experts = W.shape[0]
    b2 = b.reshape(1, num_experts)
    return pl.pallas_call(
        _gating_kernel,
        in_specs=[
            pl.BlockSpec(memory_space=pltpu.MemorySpace.HBM),
            pl.BlockSpec(memory_space=pltpu.MemorySpace.VMEM),
            pl.BlockSpec(memory_space=pltpu.MemorySpace.VMEM),
        ],
        out_specs=pl.BlockSpec(memory_space=pltpu.MemorySpace.HBM),
        out_shape=jax.ShapeDtypeStruct((n_tokens, num_experts), jnp.float32),
        scratch_shapes=[
            pltpu.VMEM((_NBUF, _TILE, input_dim), jnp.float32),
            pltpu.VMEM((_NBUF, _TILE, num_experts), jnp.float32),
            pltpu.SemaphoreType.DMA((_NBUF,)),
            pltpu.SemaphoreType.DMA((_NBUF,)),
        ],
    )(x, W, b2)


# speedup vs baseline: 1.0858x; 1.0828x over previous
import jax
import jax.numpy as jnp
from jax.experimental import pallas as pl
from jax.experimental.pallas import tpu as pltpu

_TILE = 1024


def _gating_kernel(x_ref, w_ref, b_ref, out_ref):
    logits = jax.lax.dot_general(
        x_ref[...], w_ref[...],
        dimension_numbers=(((1,), (1,)), ((), ())),
        preferred_element_type=jnp.float32,
    )
    logits = logits + b_ref[...]
    m = jnp.max(logits, axis=-1, keepdims=True)
    e = jnp.exp(logits - m)
    s = jnp.sum(e, axis=-1, keepdims=True)
    out_ref[...] = e / s


def kernel(x, W, b):
    n_tokens, input_dim = x.shape
    num_experts = W.shape[0]
    b2 = b.reshape(1, num_experts)
    return pl.pallas_call(
        _gating_kernel,
        grid=(n_tokens // _TILE,),
        in_specs=[
            pl.BlockSpec((_TILE, input_dim), lambda i: (i, 0)),
            pl.BlockSpec((num_experts, input_dim), lambda i: (0, 0)),
            pl.BlockSpec((1, num_experts), lambda i: (0, 0)),
        ],
        out_specs=pl.BlockSpec((_TILE, num_experts), lambda i: (i, 0)),
        out_shape=jax.ShapeDtypeStruct((n_tokens, num_experts), jnp.float32),
        compiler_params=pltpu.CompilerParams(
            dimension_semantics=("parallel",),
        ),
    )(x, W, b2)
